# Initial kernel scaffold; baseline (speedup 1.0000x reference)
import functools

import jax
import jax.numpy as jnp
from jax import lax
from jax.experimental import pallas as pl
from jax.experimental.pallas import tpu as pltpu
from jax.experimental.pallas import tpu_sc as plsc

N_NODES_C = 10000
D = 128
NC = 2
NS = 16
NW = NC * NS
CH = 128
R_ACC = 10016
ROWS_PER_TILE = R_ACC // NS


def _sc_segment_sum(feature, src2d, dst2d, zacc, zdeg, ones_hbm, per_w):
    mesh = plsc.VectorSubcoreMesh(core_axis_name="c", subcore_axis_name="s")

    @functools.partial(
        pl.kernel,
        out_type=(
            jax.ShapeDtypeStruct((NC, R_ACC, D), jnp.float32),
            jax.ShapeDtypeStruct((NC, R_ACC, 8), jnp.float32),
        ),
        mesh=mesh,
        scratch_types=[
            pltpu.VMEM((per_w, CH), jnp.int32),
            pltpu.VMEM((per_w, CH), jnp.int32),
            pltpu.VMEM((CH, D), jnp.float32),
            pltpu.VMEM((CH, 8), jnp.float32),
            pltpu.VMEM_SHARED((R_ACC, D), jnp.float32),
            pltpu.VMEM_SHARED((R_ACC, 8), jnp.float32),
            pltpu.SemaphoreType.DMA,
        ],
    )
    def sc_kernel(feat_hbm, src_hbm, dst_hbm, zacc_hbm, zdeg_hbm, ones_in,
                  acc_out, deg_out,
                  src_v, dst_v, rows_v, ones_v, acc_sp, deg_sp, sem):
        c = lax.axis_index("c")
        s = lax.axis_index("s")
        wid = c * NS + s
        r0 = s * ROWS_PER_TILE

        pltpu.sync_copy(zacc_hbm.at[pl.ds(r0, ROWS_PER_TILE)],
                        acc_sp.at[pl.ds(r0, ROWS_PER_TILE)])
        pltpu.sync_copy(zdeg_hbm.at[pl.ds(r0, ROWS_PER_TILE)],
                        deg_sp.at[pl.ds(r0, ROWS_PER_TILE)])
        pltpu.sync_copy(ones_in, ones_v)

        base = wid * per_w
        pltpu.sync_copy(src_hbm.at[pl.ds(base, per_w)], src_v)
        pltpu.sync_copy(dst_hbm.at[pl.ds(base, per_w)], dst_v)

        plsc.subcore_barrier()

        @pl.loop(0, per_w)
        def _(j):
            pltpu.async_copy(feat_hbm.at[src_v.at[j]], rows_v, sem).wait()
            pltpu.sync_copy(rows_v, acc_sp.at[dst_v.at[j]], add=True)
            pltpu.sync_copy(ones_v, deg_sp.at[dst_v.at[j]], add=True)

        plsc.subcore_barrier()

        pltpu.sync_copy(acc_sp.at[pl.ds(r0, ROWS_PER_TILE)],
                        acc_out.at[c, pl.ds(r0, ROWS_PER_TILE)])
        pltpu.sync_copy(deg_sp.at[pl.ds(r0, ROWS_PER_TILE)],
                        deg_out.at[c, pl.ds(r0, ROWS_PER_TILE)])

    return sc_kernel(feature, src2d, dst2d, zacc, zdeg, ones_hbm)


def _tc_finish_body(acc_ref, deg_ref, feat_ref, w_ref, b_ref, out_ref):
    summed = acc_ref[0] + acc_ref[1]
    deg = (deg_ref[0] + deg_ref[1])[:, 0:1]
    mean = summed / jnp.maximum(deg, 1.0)
    h = jnp.where(deg > 0.0, mean, feat_ref[...])
    y = lax.dot_general(h, w_ref[...], (((1,), (1,)), ((), ())),
                        preferred_element_type=jnp.float32)
    out_ref[...] = jnp.maximum(y + b_ref[...], 0.0)


def _tc_finish(acc_p, deg_p, feature, W, b2):
    blk = 1000
    grid = (N_NODES_C // blk,)
    return pl.pallas_call(
        _tc_finish_body,
        grid=grid,
        in_specs=[
            pl.BlockSpec((NC, blk, D), lambda i: (0, i, 0)),
            pl.BlockSpec((NC, blk, 8), lambda i: (0, i, 0)),
            pl.BlockSpec((blk, D), lambda i: (i, 0)),
            pl.BlockSpec((D, D), lambda i: (0, 0)),
            pl.BlockSpec((1, D), lambda i: (0, 0)),
        ],
        out_specs=pl.BlockSpec((blk, D), lambda i: (i, 0)),
        out_shape=jax.ShapeDtypeStruct((N_NODES_C, D), jnp.float32),
    )(acc_p, deg_p, feature, W, b2)


def kernel(feature, edge_index, W, b):
    n_edges = edge_index.shape[1]
    per_w = -(-n_edges // (NW * CH))
    e_pad = NW * CH * per_w
    pad = e_pad - n_edges

    src = edge_index[0]
    dst = edge_index[1]
    if pad:
        src = jnp.concatenate([src, jnp.zeros((pad,), jnp.int32)])
        dst = jnp.concatenate([dst, jnp.full((pad,), N_NODES_C, jnp.int32)])
    src2d = src.reshape(NW * per_w, CH)
    dst2d = dst.reshape(NW * per_w, CH)

    zacc = jnp.zeros((R_ACC, D), jnp.float32)
    zdeg = jnp.zeros((R_ACC, 8), jnp.float32)
    ones_hbm = jnp.ones((CH, 8), jnp.float32)

    acc_p, deg_p = _sc_segment_sum(feature, src2d, dst2d, zacc, zdeg,
                                   ones_hbm, per_w)
    return _tc_finish(acc_p, deg_p, feature, W, b.reshape(1, D))


# speedup vs baseline: 2.6875x; 2.6875x over previous
import functools

import jax
import jax.numpy as jnp
from jax import lax
from jax.experimental import pallas as pl
from jax.experimental.pallas import tpu as pltpu
from jax.experimental.pallas import tpu_sc as plsc

N_NODES_C = 10000
D = 128
NC = 2
NS = 16
CH = 128
R_ACC = 10112
ROWS_PER_TILE = R_ACC // NS


def _sc_segment_sum(feature, src3d, dst3d, zacc, ones128, per_w):
    mesh = plsc.VectorSubcoreMesh(core_axis_name="c", subcore_axis_name="s")

    @functools.partial(
        pl.kernel,
        out_type=jax.ShapeDtypeStruct((NC, R_ACC, D), jnp.float32),
        mesh=mesh,
        scratch_types=[
            pltpu.VMEM((8, CH), jnp.int32),
            pltpu.VMEM((8, CH), jnp.int32),
            pltpu.VMEM((CH, D), jnp.float32),
            pltpu.VMEM((CH, D), jnp.float32),
            pltpu.VMEM_SHARED((R_ACC, D), jnp.float32),
            pltpu.SemaphoreType.DMA,
        ],
    )
    def sc_kernel(feat_hbm, src_hbm, dst_hbm, zacc_hbm, ones_hbm,
                  acc_out,
                  src_v, dst_v, rows_v, ones_v, acc_sp, sem):
        c = lax.axis_index("c")
        s = lax.axis_index("s")
        r0 = s * ROWS_PER_TILE

        pltpu.sync_copy(zacc_hbm.at[pl.ds(r0, ROWS_PER_TILE)],
                        acc_sp.at[pl.ds(r0, ROWS_PER_TILE)])
        pltpu.sync_copy(ones_hbm, ones_v)
        plsc.subcore_barrier()

        @pl.loop(0, per_w // 8)
        def _(jo):
            pltpu.sync_copy(dst_hbm.at[s, pl.ds(jo * 8, 8)], dst_v)

            @pl.when(c == 0)
            def _():
                pltpu.sync_copy(src_hbm.at[s, pl.ds(jo * 8, 8)], src_v)

            @pl.loop(0, 8)
            def _(ji):
                @pl.when(c == 0)
                def _():
                    pltpu.async_copy(feat_hbm.at[src_v.at[ji]], rows_v,
                                     sem).wait()
                    pltpu.sync_copy(rows_v, acc_sp.at[dst_v.at[ji]], add=True)

                @pl.when(c == 1)
                def _():
                    pltpu.sync_copy(ones_v, acc_sp.at[dst_v.at[ji]], add=True)

        plsc.subcore_barrier()
        pltpu.sync_copy(acc_sp.at[pl.ds(r0, ROWS_PER_TILE)],
                        acc_out.at[c, pl.ds(r0, ROWS_PER_TILE)])

    return sc_kernel(feature, src3d, dst3d, zacc, ones128)


def _tc_finish_body(acc_ref, feat_ref, w_ref, b_ref, out_ref):
    summed = acc_ref[0]
    deg = acc_ref[1][:, 0:1]
    mean = summed / jnp.maximum(deg, 1.0)
    h = jnp.where(deg > 0.0, mean, feat_ref[...])
    y = lax.dot_general(h, w_ref[...], (((1,), (1,)), ((), ())),
                        preferred_element_type=jnp.float32)
    out_ref[...] = jnp.maximum(y + b_ref[...], 0.0)


def _tc_finish(acc_p, feature, W, b2):
    blk = 1000
    grid = (N_NODES_C // blk,)
    return pl.pallas_call(
        _tc_finish_body,
        grid=grid,
        in_specs=[
            pl.BlockSpec((NC, blk, D), lambda i: (0, i, 0)),
            pl.BlockSpec((blk, D), lambda i: (i, 0)),
            pl.BlockSpec((D, D), lambda i: (0, 0)),
            pl.BlockSpec((1, D), lambda i: (0, 0)),
        ],
        out_specs=pl.BlockSpec((blk, D), lambda i: (i, 0)),
        out_shape=jax.ShapeDtypeStruct((N_NODES_C, D), jnp.float32),
    )(acc_p, feature, W, b2)


def kernel(feature, edge_index, W, b):
    n_edges = edge_index.shape[1]
    per_w = -(-n_edges // (NS * CH))
    per_w = -(-per_w // 8) * 8
    e_pad = NS * CH * per_w
    pad = e_pad - n_edges

    src = edge_index[0]
    dst = edge_index[1]
    if pad:
        src = jnp.concatenate([src, jnp.zeros((pad,), jnp.int32)])
        dst = jnp.concatenate([dst, jnp.full((pad,), N_NODES_C, jnp.int32)])
    src3d = src.reshape(NS, per_w, CH)
    dst3d = dst.reshape(NS, per_w, CH)

    zacc = jnp.zeros((R_ACC, D), jnp.float32)
    ones128 = jnp.ones((CH, D), jnp.float32)

    acc_p = _sc_segment_sum(feature, src3d, dst3d, zacc, ones128, per_w)
    return _tc_finish(acc_p, feature, W, b.reshape(1, D))


# speedup vs baseline: 2.9687x; 1.1046x over previous
import functools

import jax
import jax.numpy as jnp
from jax import lax
from jax.experimental import pallas as pl
from jax.experimental.pallas import tpu as pltpu
from jax.experimental.pallas import tpu_sc as plsc

N_NODES_C = 10000
D = 128
NC = 2
NS = 16
CH = 128
R_ACC = 10112
ROWS_PER_TILE = R_ACC // NS


def _sc_segment_sum(feature, src3d, dst3d, zacc, ones128, per_w):
    mesh = plsc.VectorSubcoreMesh(core_axis_name="c", subcore_axis_name="s")

    @functools.partial(
        pl.kernel,
        out_type=jax.ShapeDtypeStruct((NC, R_ACC, D), jnp.float32),
        mesh=mesh,
        scratch_types=[
            pltpu.VMEM((8, CH), jnp.int32),
            pltpu.VMEM((8, CH), jnp.int32),
            pltpu.VMEM((CH, D), jnp.float32),
            pltpu.VMEM((CH, D), jnp.float32),
            pltpu.VMEM_SHARED((R_ACC, D), jnp.float32),
            pltpu.SemaphoreType.DMA,
            pltpu.SemaphoreType.DMA,
        ],
    )
    def sc_kernel(feat_hbm, src_hbm, dst_hbm, zacc_hbm, ones_hbm,
                  acc_out,
                  src_v, dst_v, rows_a, rows_b, acc_sp, sem_g, sem_s):
        c = lax.axis_index("c")
        s = lax.axis_index("s")
        r0 = s * ROWS_PER_TILE
        rows = (rows_a, rows_b)

        pltpu.sync_copy(zacc_hbm.at[pl.ds(r0, ROWS_PER_TILE)],
                        acc_sp.at[pl.ds(r0, ROWS_PER_TILE)])
        pltpu.sync_copy(ones_hbm, rows_a)
        plsc.subcore_barrier()

        @pl.when(c == 0)
        def _():
            @pl.loop(0, per_w // 8)
            def _(jo):
                pltpu.sync_copy(dst_hbm.at[s, pl.ds(jo * 8, 8)], dst_v)
                pltpu.sync_copy(src_hbm.at[s, pl.ds(jo * 8, 8)], src_v)
                g = [None, None]
                sc = [None, None]
                g[0] = pltpu.async_copy(feat_hbm.at[src_v.at[0]], rows[0],
                                        sem_g)
                for ji in range(8):
                    b = ji % 2
                    nb = (ji + 1) % 2
                    g[b].wait()
                    if ji < 7:
                        if sc[nb] is not None:
                            sc[nb].wait()
                        g[nb] = pltpu.async_copy(feat_hbm.at[src_v.at[ji + 1]],
                                                 rows[nb], sem_g)
                    sc[b] = pltpu.async_copy(rows[b], acc_sp.at[dst_v.at[ji]],
                                             sem_s, add=True)
                sc[0].wait()
                sc[1].wait()

        @pl.when(c == 1)
        def _():
            @pl.loop(0, per_w // 8)
            def _(jo):
                pltpu.sync_copy(dst_hbm.at[s, pl.ds(jo * 8, 8)], dst_v)
                scs = []
                for ji in range(8):
                    scs.append(pltpu.async_copy(rows_a,
                                                acc_sp.at[dst_v.at[ji]],
                                                sem_s, add=True))
                for cp in scs:
                    cp.wait()

        plsc.subcore_barrier()
        pltpu.sync_copy(acc_sp.at[pl.ds(r0, ROWS_PER_TILE)],
                        acc_out.at[c, pl.ds(r0, ROWS_PER_TILE)])

    return sc_kernel(feature, src3d, dst3d, zacc, ones128)


def _tc_finish_body(acc_ref, feat_ref, w_ref, b_ref, out_ref):
    summed = acc_ref[0]
    deg = acc_ref[1][:, 0:1]
    mean = summed / jnp.maximum(deg, 1.0)
    h = jnp.where(deg > 0.0, mean, feat_ref[...])
    y = lax.dot_general(h, w_ref[...], (((1,), (1,)), ((), ())),
                        preferred_element_type=jnp.float32)
    out_ref[...] = jnp.maximum(y + b_ref[...], 0.0)


def _tc_finish(acc_p, feature, W, b2):
    blk = 1000
    grid = (N_NODES_C // blk,)
    return pl.pallas_call(
        _tc_finish_body,
        grid=grid,
        in_specs=[
            pl.BlockSpec((NC, blk, D), lambda i: (0, i, 0)),
            pl.BlockSpec((blk, D), lambda i: (i, 0)),
            pl.BlockSpec((D, D), lambda i: (0, 0)),
            pl.BlockSpec((1, D), lambda i: (0, 0)),
        ],
        out_specs=pl.BlockSpec((blk, D), lambda i: (i, 0)),
        out_shape=jax.ShapeDtypeStruct((N_NODES_C, D), jnp.float32),
    )(acc_p, feature, W, b2)


def kernel(feature, edge_index, W, b):
    n_edges = edge_index.shape[1]
    per_w = -(-n_edges // (NS * CH))
    per_w = -(-per_w // 8) * 8
    e_pad = NS * CH * per_w
    pad = e_pad - n_edges

    src = edge_index[0]
    dst = edge_index[1]
    if pad:
        src = jnp.concatenate([src, jnp.zeros((pad,), jnp.int32)])
        dst = jnp.concatenate([dst, jnp.full((pad,), N_NODES_C, jnp.int32)])
    src3d = src.reshape(NS, per_w, CH)
    dst3d = dst.reshape(NS, per_w, CH)

    zacc = jnp.zeros((R_ACC, D), jnp.float32)
    ones128 = jnp.ones((CH, D), jnp.float32)

    acc_p = _sc_segment_sum(feature, src3d, dst3d, zacc, ones128, per_w)
    return _tc_finish(acc_p, feature, W, b.reshape(1, D))


# speedup vs baseline: 3.2133x; 1.0824x over previous
import functools

import jax
import jax.numpy as jnp
from jax import lax
from jax.experimental import pallas as pl
from jax.experimental.pallas import tpu as pltpu
from jax.experimental.pallas import tpu_sc as plsc

N_NODES_C = 10000
D = 128
NC = 2
NS = 16
CH = 64
GRP = 16
NB = 4
R_ACC = 10112
ROWS_PER_TILE = R_ACC // NS


def _sc_segment_sum(feature, src3d, dst3d, zacc, ones_h, per_w):
    mesh = plsc.VectorSubcoreMesh(core_axis_name="c", subcore_axis_name="s")

    @functools.partial(
        pl.kernel,
        out_type=jax.ShapeDtypeStruct((NC, R_ACC, D), jnp.float32),
        mesh=mesh,
        scratch_types=[
            pltpu.VMEM((GRP, CH), jnp.int32),
            pltpu.VMEM((GRP, CH), jnp.int32),
            pltpu.VMEM((CH, D), jnp.float32),
            pltpu.VMEM((CH, D), jnp.float32),
            pltpu.VMEM((CH, D), jnp.float32),
            pltpu.VMEM((CH, D), jnp.float32),
            pltpu.VMEM_SHARED((R_ACC, D), jnp.float32),
            pltpu.SemaphoreType.DMA,
            pltpu.SemaphoreType.DMA,
        ],
    )
    def sc_kernel(feat_hbm, src_hbm, dst_hbm, zacc_hbm, ones_hbm,
                  acc_out,
                  src_v, dst_v, rows_a, rows_b, rows_c, rows_d,
                  acc_sp, sem_g, sem_s):
        c = lax.axis_index("c")
        s = lax.axis_index("s")
        r0 = s * ROWS_PER_TILE
        rows = (rows_a, rows_b, rows_c, rows_d)

        pltpu.sync_copy(zacc_hbm.at[pl.ds(r0, ROWS_PER_TILE)],
                        acc_sp.at[pl.ds(r0, ROWS_PER_TILE)])
        pltpu.sync_copy(ones_hbm, rows_a)
        plsc.subcore_barrier()

        @pl.when(c == 0)
        def _():
            @pl.loop(0, per_w // GRP)
            def _(jo):
                pltpu.sync_copy(dst_hbm.at[s, pl.ds(jo * GRP, GRP)], dst_v)
                pltpu.sync_copy(src_hbm.at[s, pl.ds(jo * GRP, GRP)], src_v)
                g = [None] * NB
                sc = [None] * NB
                for k in range(NB - 1):
                    g[k] = pltpu.async_copy(feat_hbm.at[src_v.at[k]],
                                            rows[k], sem_g)
                for k in range(GRP):
                    b = k % NB
                    g[b].wait()
                    sc[b] = pltpu.async_copy(rows[b], acc_sp.at[dst_v.at[k]],
                                             sem_s, add=True)
                    if k + NB - 1 < GRP:
                        nb = (k + NB - 1) % NB
                        if sc[nb] is not None:
                            sc[nb].wait()
                        g[nb] = pltpu.async_copy(
                            feat_hbm.at[src_v.at[k + NB - 1]], rows[nb],
                            sem_g)
                for k in range(GRP - NB, GRP):
                    sc[k % NB].wait()

        @pl.when(c == 1)
        def _():
            @pl.loop(0, per_w // GRP)
            def _(jo):
                pltpu.sync_copy(dst_hbm.at[s, pl.ds(jo * GRP, GRP)], dst_v)
                scs = []
                for k in range(GRP):
                    scs.append(pltpu.async_copy(rows_a,
                                                acc_sp.at[dst_v.at[k]],
                                                sem_s, add=True))
                for cp in scs:
                    cp.wait()

        plsc.subcore_barrier()
        pltpu.sync_copy(acc_sp.at[pl.ds(r0, ROWS_PER_TILE)],
                        acc_out.at[c, pl.ds(r0, ROWS_PER_TILE)])

    return sc_kernel(feature, src3d, dst3d, zacc, ones_h)


def _tc_finish_body(acc_ref, feat_ref, w_ref, b_ref, out_ref):
    summed = acc_ref[0]
    deg = acc_ref[1][:, 0:1]
    mean = summed / jnp.maximum(deg, 1.0)
    h = jnp.where(deg > 0.0, mean, feat_ref[...])
    y = lax.dot_general(h, w_ref[...], (((1,), (1,)), ((), ())),
                        preferred_element_type=jnp.float32)
    out_ref[...] = jnp.maximum(y + b_ref[...], 0.0)


def _tc_finish(acc_p, feature, W, b2):
    blk = 1000
    grid = (N_NODES_C // blk,)
    return pl.pallas_call(
        _tc_finish_body,
        grid=grid,
        in_specs=[
            pl.BlockSpec((NC, blk, D), lambda i: (0, i, 0)),
            pl.BlockSpec((blk, D), lambda i: (i, 0)),
            pl.BlockSpec((D, D), lambda i: (0, 0)),
            pl.BlockSpec((1, D), lambda i: (0, 0)),
        ],
        out_specs=pl.BlockSpec((blk, D), lambda i: (i, 0)),
        out_shape=jax.ShapeDtypeStruct((N_NODES_C, D), jnp.float32),
    )(acc_p, feature, W, b2)


def kernel(feature, edge_index, W, b):
    n_edges = edge_index.shape[1]
    per_w = -(-n_edges // (NS * CH))
    per_w = -(-per_w // GRP) * GRP
    e_pad = NS * CH * per_w
    pad = e_pad - n_edges

    src = edge_index[0]
    dst = edge_index[1]
    if pad:
        src = jnp.concatenate([src, jnp.zeros((pad,), jnp.int32)])
        dst = jnp.concatenate([dst, jnp.full((pad,), N_NODES_C, jnp.int32)])
    src3d = src.reshape(NS, per_w, CH)
    dst3d = dst.reshape(NS, per_w, CH)

    zacc = jnp.zeros((R_ACC, D), jnp.float32)
    ones_h = jnp.ones((CH, D), jnp.float32)

    acc_p = _sc_segment_sum(feature, src3d, dst3d, zacc, ones_h, per_w)
    return _tc_finish(acc_p, feature, W, b.reshape(1, D))


# speedup vs baseline: 3.7411x; 1.1643x over previous
import dataclasses
import functools

import jax
import jax.numpy as jnp
from jax import lax
from jax.experimental import pallas as pl
from jax.experimental.pallas import tpu as pltpu
from jax.experimental.pallas import tpu_sc as plsc

N_NODES_C = 10000
D = 128
NC = 2
NS = 16
NW = NC * NS
CH = 64
GRP = 16
NB = 3
LANES = 16
R_ACC = 10240
ROWS_PER_TILE = R_ACC // NS


def _gather16(x, idx):
    return lax.gather(
        x, idx[:, None],
        lax.GatherDimensionNumbers(offset_dims=(), collapsed_slice_dims=(0,),
                                   start_index_map=(0,)),
        (1,), mode=lax.GatherScatterMode.PROMISE_IN_BOUNDS)


def _sc_segment_sum(feature, src3d, dst3d, zacc, per_w):
    mesh = plsc.VectorSubcoreMesh(core_axis_name="c", subcore_axis_name="s")
    cp = pltpu.CompilerParams()
    if "needs_layout_passes" in pltpu.CompilerParams.__dataclass_fields__:
        cp = dataclasses.replace(cp, needs_layout_passes=False)

    @functools.partial(
        pl.kernel,
        compiler_params=cp,
        out_type=(
            jax.ShapeDtypeStruct((NC, R_ACC, D), jnp.float32),
            jax.ShapeDtypeStruct((NW, R_ACC), jnp.float32),
        ),
        mesh=mesh,
        scratch_types=[
            pltpu.VMEM((GRP, CH), jnp.int32),
            pltpu.VMEM((GRP, CH), jnp.int32),
            pltpu.VMEM((CH, D), jnp.float32),
            pltpu.VMEM((CH, D), jnp.float32),
            pltpu.VMEM((CH, D), jnp.float32),
            pltpu.VMEM((R_ACC,), jnp.float32),
            pltpu.VMEM_SHARED((R_ACC, D), jnp.float32),
            pltpu.SemaphoreType.DMA,
            pltpu.SemaphoreType.DMA,
        ],
    )
    def sc_kernel(feat_hbm, src_hbm, dst_hbm, zacc_hbm,
                  acc_out, deg_out,
                  src_v, dst_v, rows_a, rows_b, rows_c, hist_v,
                  acc_sp, sem_g, sem_s):
        c = lax.axis_index("c")
        s = lax.axis_index("s")
        wid = c * NS + s
        r0 = s * ROWS_PER_TILE
        rows = (rows_a, rows_b, rows_c)

        pltpu.sync_copy(zacc_hbm.at[pl.ds(r0, ROWS_PER_TILE)],
                        acc_sp.at[pl.ds(r0, ROWS_PER_TILE)])

        z16 = jnp.zeros((LANES,), jnp.float32)

        @pl.loop(0, R_ACC // LANES)
        def _(i):
            hist_v[pl.ds(i * LANES, LANES)] = z16

        plsc.subcore_barrier()

        ones16 = jnp.ones((LANES,), jnp.float32)
        lane_iota = lax.iota(jnp.int32, LANES)

        @pl.loop(0, per_w // GRP)
        def _(jo):
            pltpu.sync_copy(dst_hbm.at[wid, pl.ds(jo * GRP, GRP)], dst_v)
            pltpu.sync_copy(src_hbm.at[wid, pl.ds(jo * GRP, GRP)], src_v)

            @pl.loop(0, GRP)
            def _(k):
                for g in range(CH // LANES):
                    idx = dst_v[k, pl.ds(g * LANES, LANES)]
                    sk, _ = plsc.sort_key_val(idx, idx)
                    prev = _gather16(sk, jnp.maximum(lane_iota - 1, 0))
                    first = (lane_iota == 0) | (sk != prev)
                    t = jnp.where(first, lane_iota, LANES)
                    tp1 = _gather16(t, jnp.minimum(lane_iota + 1, LANES - 1))
                    tp1 = jnp.where(lane_iota == LANES - 1, LANES, tp1)
                    sfx = -lax.rev(plsc.cummax(lax.rev(-tp1, (0,))), (0,))
                    cnt = (sfx - lane_iota).astype(jnp.float32)
                    plsc.addupdate_scatter(hist_v, [sk], cnt, mask=first)

            g = [None] * NB
            for k in range(NB - 1):
                g[k] = pltpu.async_copy(feat_hbm.at[src_v.at[k]],
                                        rows[k], sem_g)
            for k in range(GRP):
                b = k % NB
                g[b].wait()
                if k + NB - 1 < GRP:
                    nb = (k + NB - 1) % NB
                    g[nb] = pltpu.async_copy(
                        feat_hbm.at[src_v.at[k + NB - 1]], rows[nb], sem_g)
                pltpu.sync_copy(rows[b], acc_sp.at[dst_v.at[k]], add=True)

        plsc.subcore_barrier()
        pltpu.sync_copy(acc_sp.at[pl.ds(r0, ROWS_PER_TILE)],
                        acc_out.at[c, pl.ds(r0, ROWS_PER_TILE)])
        pltpu.sync_copy(hist_v, deg_out.at[wid])

    return sc_kernel(feature, src3d, dst3d, zacc)


def _tc_finish_body(acc_ref, deg_ref, feat_ref, w_ref, b_ref, out_ref):
    summed = acc_ref[0] + acc_ref[1]
    deg = lax.dot_general(deg_ref[...], jnp.ones((NW, 1), jnp.float32),
                          (((0,), (0,)), ((), ())),
                          preferred_element_type=jnp.float32)
    mean = summed / jnp.maximum(deg, 1.0)
    h = jnp.where(deg > 0.0, mean, feat_ref[...])
    y = lax.dot_general(h, w_ref[...], (((1,), (1,)), ((), ())),
                        preferred_element_type=jnp.float32)
    out_ref[...] = jnp.maximum(y + b_ref[...], 0.0)


def _tc_finish(acc_p, deg_p, feature, W, b2):
    blk = 1024
    grid = (R_ACC // blk,)
    return pl.pallas_call(
        _tc_finish_body,
        grid=grid,
        in_specs=[
            pl.BlockSpec((NC, blk, D), lambda i: (0, i, 0)),
            pl.BlockSpec((NW, blk), lambda i: (0, i)),
            pl.BlockSpec((blk, D), lambda i: (i, 0)),
            pl.BlockSpec((D, D), lambda i: (0, 0)),
            pl.BlockSpec((1, D), lambda i: (0, 0)),
        ],
        out_specs=pl.BlockSpec((blk, D), lambda i: (i, 0)),
        out_shape=jax.ShapeDtypeStruct((R_ACC, D), jnp.float32),
    )(acc_p, deg_p, feature, W, b2)


def kernel(feature, edge_index, W, b):
    n_edges = edge_index.shape[1]
    per_w = -(-n_edges // (NW * CH))
    per_w = -(-per_w // GRP) * GRP
    e_pad = NW * CH * per_w
    pad = e_pad - n_edges

    src = edge_index[0]
    dst = edge_index[1]
    if pad:
        src = jnp.concatenate([src, jnp.zeros((pad,), jnp.int32)])
        dst = jnp.concatenate([dst, jnp.full((pad,), N_NODES_C, jnp.int32)])
    src3d = src.reshape(NW, per_w, CH)
    dst3d = dst.reshape(NW, per_w, CH)

    zacc = jnp.zeros((R_ACC, D), jnp.float32)

    acc_p, deg_p = _sc_segment_sum(feature, src3d, dst3d, zacc, per_w)
    out = _tc_finish(acc_p, deg_p, feature, W, b.reshape(1, D))
    return out[:N_NODES_C]


# speedup vs baseline: 5.5605x; 1.4863x over previous
import dataclasses
import functools

import jax
import jax.numpy as jnp
import numpy as np
from jax import lax
from jax.experimental import pallas as pl
from jax.experimental.pallas import tpu as pltpu
from jax.experimental.pallas import tpu_sc as plsc

N_NODES_C = 10000
D = 128
NC = 2
NS = 16
NW = NC * NS
CH = 64
GRP = 16
NB = 3
LANES = 16
R_ACC = 10240
ROWS_PER_TILE = R_ACC // NS


_PERM = np.empty((D,), np.int32)
for _g in range(D // 32):
    for _j in range(16):
        _PERM[_g * 32 + 2 * _j] = _g * 32 + _j
        _PERM[_g * 32 + 2 * _j + 1] = _g * 32 + 16 + _j


def _gather16(x, idx):
    return lax.gather(
        x, idx[:, None],
        lax.GatherDimensionNumbers(offset_dims=(), collapsed_slice_dims=(0,),
                                   start_index_map=(0,)),
        (1,), mode=lax.GatherScatterMode.PROMISE_IN_BOUNDS)


def _sc_segment_sum(feature, src3d, dst3d, zacc, per_w):
    mesh = plsc.VectorSubcoreMesh(core_axis_name="c", subcore_axis_name="s")
    cp = pltpu.CompilerParams()
    if "needs_layout_passes" in pltpu.CompilerParams.__dataclass_fields__:
        cp = dataclasses.replace(cp, needs_layout_passes=False)
    if "use_tc_tiling_on_sc" in pltpu.CompilerParams.__dataclass_fields__:
        cp = dataclasses.replace(cp, use_tc_tiling_on_sc=False)

    @functools.partial(
        pl.kernel,
        compiler_params=cp,
        out_type=(
            jax.ShapeDtypeStruct((NC, R_ACC, D), jnp.float32),
            jax.ShapeDtypeStruct((NW, R_ACC), jnp.float32),
        ),
        mesh=mesh,
        scratch_types=[
            pltpu.VMEM((GRP, CH), jnp.int32),
            pltpu.VMEM((GRP, CH), jnp.int32),
            pltpu.VMEM((CH, D), jnp.bfloat16),
            pltpu.VMEM((CH, D), jnp.bfloat16),
            pltpu.VMEM((CH, D), jnp.bfloat16),
            pltpu.VMEM((CH, D), jnp.float32),
            pltpu.VMEM((R_ACC,), jnp.float32),
            pltpu.VMEM_SHARED((R_ACC, D), jnp.float32),
            pltpu.SemaphoreType.DMA,
            pltpu.SemaphoreType.DMA,
        ],
    )
    def sc_kernel(feat_hbm, src_hbm, dst_hbm, zacc_hbm,
                  acc_out, deg_out,
                  src_v, dst_v, rows_a, rows_b, rows_c, conv_v, hist_v,
                  acc_sp, sem_g, sem_s):
        c = lax.axis_index("c")
        s = lax.axis_index("s")
        wid = c * NS + s
        r0 = s * ROWS_PER_TILE
        rows = (rows_a, rows_b, rows_c)

        pltpu.sync_copy(zacc_hbm.at[pl.ds(r0, ROWS_PER_TILE)],
                        acc_sp.at[pl.ds(r0, ROWS_PER_TILE)])

        z16 = jnp.zeros((LANES,), jnp.float32)

        @pl.loop(0, R_ACC // LANES)
        def _(i):
            hist_v[pl.ds(i * LANES, LANES)] = z16

        plsc.subcore_barrier()

        ones16 = jnp.ones((LANES,), jnp.float32)
        lane_iota = lax.iota(jnp.int32, LANES)

        @pl.loop(0, per_w // GRP)
        def _(jo):
            pltpu.sync_copy(dst_hbm.at[wid, pl.ds(jo * GRP, GRP)], dst_v)
            pltpu.sync_copy(src_hbm.at[wid, pl.ds(jo * GRP, GRP)], src_v)

            @pl.loop(0, GRP)
            def _(k):
                for g in range(CH // LANES):
                    idx = dst_v[k, pl.ds(g * LANES, LANES)]
                    sk, _ = plsc.sort_key_val(idx, idx)
                    prev = _gather16(sk, jnp.maximum(lane_iota - 1, 0))
                    first = (lane_iota == 0) | (sk != prev)
                    t = jnp.where(first, lane_iota, LANES)
                    tp1 = _gather16(t, jnp.minimum(lane_iota + 1, LANES - 1))
                    tp1 = jnp.where(lane_iota == LANES - 1, LANES, tp1)
                    sfx = -lax.rev(plsc.cummax(lax.rev(-tp1, (0,))), (0,))
                    cnt = (sfx - lane_iota).astype(jnp.float32)
                    plsc.addupdate_scatter(hist_v, [sk], cnt, mask=first)

            g = [None] * NB
            for k in range(NB - 1):
                g[k] = pltpu.async_copy(feat_hbm.at[src_v.at[k]],
                                        rows[k], sem_g)
            for k in range(GRP):
                b = k % NB
                g[b].wait()
                if k + NB - 1 < GRP:
                    nb = (k + NB - 1) % NB
                    g[nb] = pltpu.async_copy(
                        feat_hbm.at[src_v.at[k + NB - 1]], rows[nb], sem_g)

                rb = rows[b]
                hmask = jnp.uint32(0xFFFF0000)

                @pl.loop(0, CH)
                def _(r):
                    for g2 in range(D // 32):
                        v = rb[r, pl.ds(g2 * 32, 32)]
                        u = plsc.bitcast(v, jnp.uint32)
                        lo = plsc.bitcast(u << 16, jnp.float32)
                        hi = plsc.bitcast(u & hmask, jnp.float32)
                        conv_v[r, pl.ds(g2 * 32, LANES)] = lo
                        conv_v[r, pl.ds(g2 * 32 + LANES, LANES)] = hi

                pltpu.sync_copy(conv_v, acc_sp.at[dst_v.at[k]], add=True)

        plsc.subcore_barrier()
        pltpu.sync_copy(acc_sp.at[pl.ds(r0, ROWS_PER_TILE)],
                        acc_out.at[c, pl.ds(r0, ROWS_PER_TILE)])
        pltpu.sync_copy(hist_v, deg_out.at[wid])

    return sc_kernel(feature, src3d, dst3d, zacc)


def _tc_finish_body(acc_ref, deg_ref, feat_ref, w_ref, b_ref, out_ref):
    summed = acc_ref[0] + acc_ref[1]
    deg = lax.dot_general(deg_ref[...], jnp.ones((NW, 1), jnp.float32),
                          (((0,), (0,)), ((), ())),
                          preferred_element_type=jnp.float32)
    mean = summed / jnp.maximum(deg, 1.0)
    h = jnp.where(deg > 0.0, mean, feat_ref[...])
    y = lax.dot_general(h, w_ref[...], (((1,), (1,)), ((), ())),
                        preferred_element_type=jnp.float32)
    out_ref[...] = jnp.maximum(y + b_ref[...], 0.0)


def _tc_finish(acc_p, deg_p, feature, W, b2):
    blk = 1024
    grid = (R_ACC // blk,)
    return pl.pallas_call(
        _tc_finish_body,
        grid=grid,
        in_specs=[
            pl.BlockSpec((NC, blk, D), lambda i: (0, i, 0)),
            pl.BlockSpec((NW, blk), lambda i: (0, i)),
            pl.BlockSpec((blk, D), lambda i: (i, 0)),
            pl.BlockSpec((D, D), lambda i: (0, 0)),
            pl.BlockSpec((1, D), lambda i: (0, 0)),
        ],
        out_specs=pl.BlockSpec((blk, D), lambda i: (i, 0)),
        out_shape=jax.ShapeDtypeStruct((R_ACC, D), jnp.float32),
    )(acc_p, deg_p, feature, W, b2)


def kernel(feature, edge_index, W, b):
    n_edges = edge_index.shape[1]
    per_w = -(-n_edges // (NW * CH))
    per_w = -(-per_w // GRP) * GRP
    e_pad = NW * CH * per_w
    pad = e_pad - n_edges

    src = edge_index[0]
    dst = edge_index[1]
    if pad:
        src = jnp.concatenate([src, jnp.zeros((pad,), jnp.int32)])
        dst = jnp.concatenate([dst, jnp.full((pad,), N_NODES_C, jnp.int32)])
    src3d = src.reshape(NW, per_w, CH)
    dst3d = dst.reshape(NW, per_w, CH)

    zacc = jnp.zeros((R_ACC, D), jnp.float32)
    feat_tab = feature[:, _PERM].astype(jnp.bfloat16)

    acc_p, deg_p = _sc_segment_sum(feat_tab, src3d, dst3d, zacc, per_w)
    out = _tc_finish(acc_p, deg_p, feature, W, b.reshape(1, D))
    return out[:N_NODES_C]


# speedup vs baseline: 5.5665x; 1.0011x over previous
import dataclasses
import functools

import jax
import jax.numpy as jnp
import numpy as np
from jax import lax
from jax.experimental import pallas as pl
from jax.experimental.pallas import tpu as pltpu
from jax.experimental.pallas import tpu_sc as plsc

N_NODES_C = 10000
D = 128
NC = 2
NS = 16
NW = NC * NS
CH = 64
GRP = 16
NB = 4
LANES = 16
R_ACC = 10240
ROWS_PER_TILE = R_ACC // NS


_PERM = np.empty((D,), np.int32)
for _g in range(D // 32):
    for _j in range(16):
        _PERM[_g * 32 + 2 * _j] = _g * 32 + _j
        _PERM[_g * 32 + 2 * _j + 1] = _g * 32 + 16 + _j


def _gather16(x, idx):
    return lax.gather(
        x, idx[:, None],
        lax.GatherDimensionNumbers(offset_dims=(), collapsed_slice_dims=(0,),
                                   start_index_map=(0,)),
        (1,), mode=lax.GatherScatterMode.PROMISE_IN_BOUNDS)


def _sc_segment_sum(feature, src3d, dst3d, zacc, per_w):
    mesh = plsc.VectorSubcoreMesh(core_axis_name="c", subcore_axis_name="s")
    cp = pltpu.CompilerParams()
    if "needs_layout_passes" in pltpu.CompilerParams.__dataclass_fields__:
        cp = dataclasses.replace(cp, needs_layout_passes=False)
    if "use_tc_tiling_on_sc" in pltpu.CompilerParams.__dataclass_fields__:
        cp = dataclasses.replace(cp, use_tc_tiling_on_sc=False)

    @functools.partial(
        pl.kernel,
        compiler_params=cp,
        out_type=(
            jax.ShapeDtypeStruct((NC, R_ACC, D), jnp.float32),
            jax.ShapeDtypeStruct((NW, R_ACC), jnp.float32),
        ),
        mesh=mesh,
        scratch_types=[
            pltpu.VMEM((GRP, CH), jnp.int32),
            pltpu.VMEM((GRP, CH), jnp.int32),
            pltpu.VMEM((CH, D), jnp.bfloat16),
            pltpu.VMEM((CH, D), jnp.bfloat16),
            pltpu.VMEM((CH, D), jnp.bfloat16),
            pltpu.VMEM((CH, D), jnp.bfloat16),
            pltpu.VMEM((CH, D), jnp.float32),
            pltpu.VMEM((R_ACC,), jnp.float32),
            pltpu.VMEM_SHARED((R_ACC, D), jnp.float32),
            pltpu.SemaphoreType.DMA,
            pltpu.SemaphoreType.DMA,
        ],
    )
    def sc_kernel(feat_hbm, src_hbm, dst_hbm, zacc_hbm,
                  acc_out, deg_out,
                  src_v, dst_v, rows_a, rows_b, rows_c, rows_d, conv_v,
                  hist_v, acc_sp, sem_g, sem_s):
        c = lax.axis_index("c")
        s = lax.axis_index("s")
        wid = c * NS + s
        r0 = s * ROWS_PER_TILE
        rows = (rows_a, rows_b, rows_c, rows_d)

        pltpu.sync_copy(zacc_hbm.at[pl.ds(r0, ROWS_PER_TILE)],
                        acc_sp.at[pl.ds(r0, ROWS_PER_TILE)])

        z16 = jnp.zeros((LANES,), jnp.float32)

        @pl.loop(0, R_ACC // LANES)
        def _(i):
            hist_v[pl.ds(i * LANES, LANES)] = z16

        plsc.subcore_barrier()

        ones16 = jnp.ones((LANES,), jnp.float32)
        lane_iota = lax.iota(jnp.int32, LANES)

        @pl.loop(0, per_w // GRP)
        def _(jo):
            pltpu.sync_copy(dst_hbm.at[wid, pl.ds(jo * GRP, GRP)], dst_v)
            pltpu.sync_copy(src_hbm.at[wid, pl.ds(jo * GRP, GRP)], src_v)

            @pl.loop(0, GRP)
            def _(k):
                for g in range(CH // LANES):
                    idx = dst_v[k, pl.ds(g * LANES, LANES)]
                    sk, _ = plsc.sort_key_val(idx, idx)
                    prev = _gather16(sk, jnp.maximum(lane_iota - 1, 0))
                    first = (lane_iota == 0) | (sk != prev)
                    t = jnp.where(first, lane_iota, LANES)
                    tp1 = _gather16(t, jnp.minimum(lane_iota + 1, LANES - 1))
                    tp1 = jnp.where(lane_iota == LANES - 1, LANES, tp1)
                    sfx = -lax.rev(plsc.cummax(lax.rev(-tp1, (0,))), (0,))
                    cnt = (sfx - lane_iota).astype(jnp.float32)
                    plsc.addupdate_scatter(hist_v, [sk], cnt, mask=first)

            g = [None] * NB
            for k in range(NB - 1):
                g[k] = pltpu.async_copy(feat_hbm.at[src_v.at[k]],
                                        rows[k], sem_g)
            for k in range(GRP):
                b = k % NB
                g[b].wait()
                if k + NB - 1 < GRP:
                    nb = (k + NB - 1) % NB
                    g[nb] = pltpu.async_copy(
                        feat_hbm.at[src_v.at[k + NB - 1]], rows[nb], sem_g)

                rb = rows[b]
                hmask = jnp.uint32(0xFFFF0000)

                @pl.loop(0, CH)
                def _(r):
                    for g2 in range(D // 32):
                        v = rb[r, pl.ds(g2 * 32, 32)]
                        u = plsc.bitcast(v, jnp.uint32)
                        lo = plsc.bitcast(u << 16, jnp.float32)
                        hi = plsc.bitcast(u & hmask, jnp.float32)
                        conv_v[r, pl.ds(g2 * 32, LANES)] = lo
                        conv_v[r, pl.ds(g2 * 32 + LANES, LANES)] = hi

                pltpu.sync_copy(conv_v, acc_sp.at[dst_v.at[k]], add=True)

        plsc.subcore_barrier()
        pltpu.sync_copy(acc_sp.at[pl.ds(r0, ROWS_PER_TILE)],
                        acc_out.at[c, pl.ds(r0, ROWS_PER_TILE)])
        pltpu.sync_copy(hist_v, deg_out.at[wid])

    return sc_kernel(feature, src3d, dst3d, zacc)


def _tc_finish_body(acc_ref, deg_ref, feat_ref, w_ref, b_ref, out_ref):
    summed = acc_ref[0] + acc_ref[1]
    deg = lax.dot_general(deg_ref[...], jnp.ones((NW, 1), jnp.float32),
                          (((0,), (0,)), ((), ())),
                          preferred_element_type=jnp.float32)
    mean = summed / jnp.maximum(deg, 1.0)
    h = jnp.where(deg > 0.0, mean, feat_ref[...])
    y = lax.dot_general(h, w_ref[...], (((1,), (1,)), ((), ())),
                        preferred_element_type=jnp.float32)
    out_ref[...] = jnp.maximum(y + b_ref[...], 0.0)


def _tc_finish(acc_p, deg_p, feature, W, b2):
    blk = 1024
    grid = (R_ACC // blk,)
    return pl.pallas_call(
        _tc_finish_body,
        grid=grid,
        in_specs=[
            pl.BlockSpec((NC, blk, D), lambda i: (0, i, 0)),
            pl.BlockSpec((NW, blk), lambda i: (0, i)),
            pl.BlockSpec((blk, D), lambda i: (i, 0)),
            pl.BlockSpec((D, D), lambda i: (0, 0)),
            pl.BlockSpec((1, D), lambda i: (0, 0)),
        ],
        out_specs=pl.BlockSpec((blk, D), lambda i: (i, 0)),
        out_shape=jax.ShapeDtypeStruct((R_ACC, D), jnp.float32),
    )(acc_p, deg_p, feature, W, b2)


def kernel(feature, edge_index, W, b):
    n_edges = edge_index.shape[1]
    per_w = -(-n_edges // (NW * CH))
    per_w = -(-per_w // GRP) * GRP
    e_pad = NW * CH * per_w
    pad = e_pad - n_edges

    src = edge_index[0]
    dst = edge_index[1]
    if pad:
        src = jnp.concatenate([src, jnp.zeros((pad,), jnp.int32)])
        dst = jnp.concatenate([dst, jnp.full((pad,), N_NODES_C, jnp.int32)])
    src3d = src.reshape(NW, per_w, CH)
    dst3d = dst.reshape(NW, per_w, CH)

    zacc = jnp.zeros((R_ACC, D), jnp.float32)
    feat_tab = feature[:, _PERM].astype(jnp.bfloat16)

    acc_p, deg_p = _sc_segment_sum(feat_tab, src3d, dst3d, zacc, per_w)
    out = _tc_finish(acc_p, deg_p, feature, W, b.reshape(1, D))
    return out[:N_NODES_C]
